# Initial kernel scaffold; baseline (speedup 1.0000x reference)
import jax
import jax.numpy as jnp
from jax.experimental import pallas as pl


def kernel(x, graph, W_l, W_r, A1, A2, bias):
    raise NotImplementedError("write your pallas kernel here")


# speedup vs baseline: 39.0468x; 39.0468x over previous
import dataclasses
import functools

import jax
import jax.numpy as jnp
from jax import lax
from jax.experimental import pallas as pl
from jax.experimental.pallas import tpu as pltpu
from jax.experimental.pallas import tpu_sc as plsc

NC = 2
NS = 16
NW = NC * NS
LANES = 16
CH = 128

_dots = (((1,), (1,)), ((), ()))


def _proj_body(x_ref, wl_ref, a1_ref, a2_ref, xl_ref, s1_ref, s2_ref):
    x = x_ref[...]
    xl_ref[...] = lax.dot_general(x, wl_ref[...], _dots, preferred_element_type=jnp.float32)
    s1_ref[...] = lax.dot_general(x, a1_ref[...], _dots, preferred_element_type=jnp.float32)
    s2_ref[...] = lax.dot_general(x, a2_ref[...], _dots, preferred_element_type=jnp.float32)


def _out_body(acc_ref, den_ref, x_ref, wr_ref, b_ref, o_ref):
    num = acc_ref[0] + acc_ref[1]
    den = den_ref[0] + den_ref[1]
    xr = lax.dot_general(x_ref[...], wr_ref[...], _dots, preferred_element_type=jnp.float32)
    o_ref[...] = num / (den + 1e-30) + xr + b_ref[...]


def _edge_body(xl_hbm, s1_hbm, s2_hbm, row_hbm, col_hbm, acc_hbm, den_hbm,
               acc_sh, den_sh, s1_v, s2_v, ri_v, ci_v, g_v, w_v, zd_v, sem,
               *, n, n_pad, d, epw):
    cid = lax.axis_index("c")
    sid = lax.axis_index("s")
    wid = cid * NS + sid

    zeros16 = jnp.zeros((LANES,), jnp.float32)

    @pl.loop(0, CH)
    def _(e):
        for k in range(d // LANES):
            g_v[e, pl.ds(k * LANES, LANES)] = zeros16

    rows_per_tile = n_pad // NS
    for k in range(rows_per_tile // LANES):
        zd_v[pl.ds(k * LANES, LANES)] = zeros16

    zbase = sid * rows_per_tile
    for z in range(rows_per_tile // CH):
        pltpu.sync_copy(g_v, acc_sh.at[pl.ds(zbase + z * CH, CH)])
    pltpu.sync_copy(zd_v, den_sh.at[pl.ds(zbase, rows_per_tile)])

    pltpu.sync_copy(s1_hbm, s1_v)
    pltpu.sync_copy(s2_hbm, s2_v)
    plsc.subcore_barrier()

    base0 = wid * epw

    @pl.loop(0, epw // CH)
    def _(j):
        base = base0 + j * CH
        pltpu.sync_copy(row_hbm.at[pl.ds(base, CH)], ri_v)
        pltpu.sync_copy(col_hbm.at[pl.ds(base, CH)], ci_v)
        pltpu.async_copy(xl_hbm.at[ci_v], g_v, sem).wait()

        @pl.loop(0, CH // LANES)
        def _(k):
            r16 = ri_v[pl.ds(k * LANES, LANES)]
            c16 = ci_v[pl.ds(k * LANES, LANES)]
            a = plsc.load_gather(s1_v, [r16]) + plsc.load_gather(s2_v, [c16])
            a = jnp.where(a >= 0.0, a, 0.2 * a)
            w16 = jnp.exp(a)
            w_v[pl.ds(k * LANES, LANES)] = w16
            for i in range(LANES):
                wv = jnp.full((LANES,), w16[i], jnp.float32)
                e = k * LANES + i
                for k2 in range(d // LANES):
                    sl = pl.ds(k2 * LANES, LANES)
                    g_v[e, sl] = g_v[e, sl] * wv

        pltpu.sync_copy(g_v, acc_sh.at[ri_v], add=True)
        pltpu.sync_copy(w_v, den_sh.at[ri_v], add=True)

    plsc.subcore_barrier()
    pltpu.sync_copy(acc_sh.at[pl.ds(zbase, rows_per_tile)],
                    acc_hbm.at[cid, pl.ds(zbase, rows_per_tile)])
    pltpu.sync_copy(den_sh.at[pl.ds(zbase, rows_per_tile)],
                    den_hbm.at[cid, pl.ds(zbase, rows_per_tile)])


def kernel(x, graph, W_l, W_r, A1, A2, bias):
    n, d = x.shape
    out_dim = W_l.shape[0]
    e = graph.shape[1]
    n_pad = ((n + LANES + NS * LANES - 1) // (NS * LANES)) * (NS * LANES)
    chunk_total = NW * CH
    e_pad = ((e + chunk_total - 1) // chunk_total) * chunk_total
    epw = e_pad // NW

    row = graph[0]
    col = graph[1]
    pad = e_pad - e
    if pad:
        pid = jnp.arange(pad, dtype=jnp.int32)
        row = jnp.concatenate([row, n + (pid % LANES)])
        col = jnp.concatenate([col, (pid * 997) % n])

    blk = 1000
    grid = n // blk
    xl, s1, s2 = pl.pallas_call(
        _proj_body,
        grid=(grid,),
        in_specs=[
            pl.BlockSpec((blk, d), lambda i: (i, 0)),
            pl.BlockSpec((out_dim, d), lambda i: (0, 0)),
            pl.BlockSpec((1, d), lambda i: (0, 0)),
            pl.BlockSpec((1, d), lambda i: (0, 0)),
        ],
        out_specs=[
            pl.BlockSpec((blk, out_dim), lambda i: (i, 0)),
            pl.BlockSpec((blk, 1), lambda i: (i, 0)),
            pl.BlockSpec((blk, 1), lambda i: (i, 0)),
        ],
        out_shape=[
            jax.ShapeDtypeStruct((n, out_dim), jnp.float32),
            jax.ShapeDtypeStruct((n, 1), jnp.float32),
            jax.ShapeDtypeStruct((n, 1), jnp.float32),
        ],
    )(x, W_l, A1, A2)

    mesh = plsc.VectorSubcoreMesh(core_axis_name="c", subcore_axis_name="s",
                                  num_cores=NC, num_subcores=NS)
    cp = pltpu.CompilerParams()
    if "needs_layout_passes" in pltpu.CompilerParams.__dataclass_fields__:
        cp = dataclasses.replace(cp, needs_layout_passes=False)
    edge_kernel = pl.kernel(
        functools.partial(_edge_body, n=n, n_pad=n_pad, d=out_dim, epw=epw),
        out_type=[
            jax.ShapeDtypeStruct((NC, n_pad, out_dim), jnp.float32),
            jax.ShapeDtypeStruct((NC, n_pad), jnp.float32),
        ],
        mesh=mesh,
        scratch_types=[
            pltpu.VMEM_SHARED((n_pad, out_dim), jnp.float32),
            pltpu.VMEM_SHARED((n_pad,), jnp.float32),
            pltpu.VMEM((n,), jnp.float32),
            pltpu.VMEM((n,), jnp.float32),
            pltpu.VMEM((CH,), jnp.int32),
            pltpu.VMEM((CH,), jnp.int32),
            pltpu.VMEM((CH, out_dim), jnp.float32),
            pltpu.VMEM((CH,), jnp.float32),
            pltpu.VMEM((n_pad // NS,), jnp.float32),
            pltpu.SemaphoreType.DMA,
        ],
        compiler_params=cp,
    )
    acc, den = edge_kernel(xl, s1.reshape(n), s2.reshape(n), row, col)

    out = pl.pallas_call(
        _out_body,
        grid=(grid,),
        in_specs=[
            pl.BlockSpec((NC, blk, out_dim), lambda i: (0, i, 0)),
            pl.BlockSpec((NC, blk, 1), lambda i: (0, i, 0)),
            pl.BlockSpec((blk, d), lambda i: (i, 0)),
            pl.BlockSpec((out_dim, d), lambda i: (0, 0)),
            pl.BlockSpec((1, out_dim), lambda i: (0, 0)),
        ],
        out_specs=pl.BlockSpec((blk, out_dim), lambda i: (i, 0)),
        out_shape=jax.ShapeDtypeStruct((n, out_dim), jnp.float32),
    )(acc, den.reshape(NC, n_pad, 1), x, W_r, bias.reshape(1, out_dim))
    return out


# speedup vs baseline: 76.1777x; 1.9509x over previous
import dataclasses
import functools

import jax
import jax.numpy as jnp
from jax import lax
from jax.experimental import pallas as pl
from jax.experimental.pallas import tpu as pltpu
from jax.experimental.pallas import tpu_sc as plsc

NC = 2
NS = 16
NW = NC * NS
LANES = 16
CH = 112

_dots = (((1,), (1,)), ((), ()))


def _proj_body(x_ref, wl_ref, a1_ref, a2_ref, xl_ref, s1_ref, s2_ref):
    x = x_ref[...]
    xl_ref[...] = lax.dot_general(x, wl_ref[...], _dots, preferred_element_type=jnp.float32)
    s1_ref[...] = lax.dot_general(x, a1_ref[...], _dots, preferred_element_type=jnp.float32)
    s2_ref[...] = lax.dot_general(x, a2_ref[...], _dots, preferred_element_type=jnp.float32)


def _out_body(acc_ref, den_ref, x_ref, wr_ref, b_ref, o_ref):
    num = acc_ref[0] + acc_ref[1]
    den = den_ref[0] + den_ref[1]
    xr = lax.dot_general(x_ref[...], wr_ref[...], _dots, preferred_element_type=jnp.float32)
    o_ref[...] = num / (den + 1e-30) + xr + b_ref[...]


def _edge_body(xl_hbm, s1_hbm, s2_hbm, row_hbm, col_hbm, acc_hbm, den_hbm,
               acc_sh, den_sh, idx_v, sv_v, g_v, w_v,
               isem, vsem, gsem, ssem, *, n, n_pad, d, epw):
    cid = lax.axis_index("c")
    sid = lax.axis_index("s")
    wid = cid * NS + sid
    nchunks = epw // CH

    zeros16 = jnp.zeros((LANES,), jnp.float32)

    @pl.loop(0, CH)
    def _(e):
        for k in range(d // LANES):
            g_v[0, e, pl.ds(k * LANES, LANES)] = zeros16
    for r in range(2):
        for k in range(CH // LANES):
            w_v[r, pl.ds(k * LANES, LANES)] = zeros16

    rows_per_tile = n_pad // NS
    zbase = sid * rows_per_tile
    nfull = rows_per_tile // CH
    for z in range(nfull):
        pltpu.sync_copy(g_v.at[0], acc_sh.at[pl.ds(zbase + z * CH, CH)])
        pltpu.sync_copy(w_v.at[0], den_sh.at[pl.ds(zbase + z * CH, CH)])
    rem = rows_per_tile - nfull * CH
    if rem:
        pltpu.sync_copy(g_v.at[0, pl.ds(0, rem)],
                        acc_sh.at[pl.ds(zbase + nfull * CH, rem)])
        pltpu.sync_copy(w_v.at[0, pl.ds(0, rem)],
                        den_sh.at[pl.ds(zbase + nfull * CH, rem)])
    plsc.subcore_barrier()

    base0 = wid * nchunks

    def idx_start(j):
        m = lax.rem(j, 4)
        pltpu.async_copy(row_hbm.at[base0 + j], idx_v.at[m], isem.at[m])
        pltpu.async_copy(col_hbm.at[base0 + j], idx_v.at[4 + m], isem.at[m])

    def idx_wait(j):
        m = lax.rem(j, 4)
        pltpu.make_async_copy(row_hbm.at[base0 + j], idx_v.at[m],
                              isem.at[m]).wait()
        pltpu.make_async_copy(col_hbm.at[base0 + j], idx_v.at[4 + m],
                              isem.at[m]).wait()

    def sv_start(j):
        m = lax.rem(j, 4)
        pltpu.async_copy(s1_hbm.at[idx_v.at[m]], sv_v.at[m], vsem.at[m])
        pltpu.async_copy(s2_hbm.at[idx_v.at[4 + m]], sv_v.at[4 + m],
                         vsem.at[m])

    def sv_wait(j):
        m = lax.rem(j, 4)
        pltpu.make_async_copy(s1_hbm.at[idx_v.at[m]], sv_v.at[m],
                              vsem.at[m]).wait()
        pltpu.make_async_copy(s2_hbm.at[idx_v.at[4 + m]], sv_v.at[4 + m],
                              vsem.at[m]).wait()

    def g_start(j):
        pltpu.async_copy(xl_hbm.at[idx_v.at[4 + lax.rem(j, 4)]],
                         g_v.at[lax.rem(j, 3)], gsem.at[lax.rem(j, 3)])

    def g_wait(j):
        pltpu.make_async_copy(xl_hbm.at[idx_v.at[4 + lax.rem(j, 4)]],
                              g_v.at[lax.rem(j, 3)],
                              gsem.at[lax.rem(j, 3)]).wait()

    def scatter_start(j):
        m3, m4, m2 = lax.rem(j, 3), lax.rem(j, 4), lax.rem(j, 2)
        pltpu.async_copy(g_v.at[m3], acc_sh.at[idx_v.at[m4]], ssem.at[m3],
                         add=True)
        pltpu.async_copy(w_v.at[m2], den_sh.at[idx_v.at[m4]], ssem.at[m3],
                         add=True)

    def scatter_wait(j):
        m3, m4, m2 = lax.rem(j, 3), lax.rem(j, 4), lax.rem(j, 2)
        pltpu.make_async_copy(g_v.at[m3], acc_sh.at[idx_v.at[m4]],
                              ssem.at[m3]).wait()
        pltpu.make_async_copy(w_v.at[m2], den_sh.at[idx_v.at[m4]],
                              ssem.at[m3]).wait()

    def compute(j):
        m4, m3, m2 = lax.rem(j, 4), lax.rem(j, 3), lax.rem(j, 2)
        for k in range(CH // LANES):
            sl = pl.ds(k * LANES, LANES)
            a = sv_v[m4, sl] + sv_v[4 + m4, sl]
            a = jnp.where(a >= 0.0, a, 0.2 * a)
            w16 = jnp.exp(a)
            w_v[m2, sl] = w16
            for i in range(LANES):
                wv = jnp.full((LANES,), w16[i], jnp.float32)
                e = k * LANES + i
                for k2 in range(d // LANES):
                    s2l = pl.ds(k2 * LANES, LANES)
                    g_v[m3, e, s2l] = g_v[m3, e, s2l] * wv

    idx_start(0)
    idx_start(1)
    idx_wait(0)
    sv_start(0)
    g_start(0)

    @pl.loop(0, nchunks)
    def _(j):
        @pl.when(j >= 2)
        def _():
            scatter_wait(j - 2)

        @pl.when(j + 1 < nchunks)
        def _():
            idx_wait(j + 1)
            sv_start(j + 1)
            g_start(j + 1)

        @pl.when(j + 2 < nchunks)
        def _():
            idx_start(j + 2)

        sv_wait(j)
        g_wait(j)
        compute(j)
        scatter_start(j)

    scatter_wait(nchunks - 2)
    scatter_wait(nchunks - 1)

    plsc.subcore_barrier()
    pltpu.sync_copy(acc_sh.at[pl.ds(zbase, rows_per_tile)],
                    acc_hbm.at[cid, pl.ds(zbase, rows_per_tile)])
    pltpu.sync_copy(den_sh.at[pl.ds(zbase, rows_per_tile)],
                    den_hbm.at[cid, pl.ds(zbase, rows_per_tile)])


def kernel(x, graph, W_l, W_r, A1, A2, bias):
    n, d = x.shape
    out_dim = W_l.shape[0]
    e = graph.shape[1]
    n_pad = ((n + LANES + NS * LANES - 1) // (NS * LANES)) * (NS * LANES)
    chunk_total = NW * CH
    e_pad = ((e + chunk_total - 1) // chunk_total) * chunk_total
    epw = e_pad // NW
    nchunks = epw // CH

    row = graph[0]
    col = graph[1]
    pad = e_pad - e
    if pad:
        pid = jnp.arange(pad, dtype=jnp.int32)
        row = jnp.concatenate([row, n + (pid % LANES)])
        col = jnp.concatenate([col, (pid * 997) % n])
    row = row.reshape(NW * nchunks, CH)
    col = col.reshape(NW * nchunks, CH)

    blk = 1000
    grid = n // blk
    xl, s1, s2 = pl.pallas_call(
        _proj_body,
        grid=(grid,),
        in_specs=[
            pl.BlockSpec((blk, d), lambda i: (i, 0)),
            pl.BlockSpec((out_dim, d), lambda i: (0, 0)),
            pl.BlockSpec((1, d), lambda i: (0, 0)),
            pl.BlockSpec((1, d), lambda i: (0, 0)),
        ],
        out_specs=[
            pl.BlockSpec((blk, out_dim), lambda i: (i, 0)),
            pl.BlockSpec((blk, 1), lambda i: (i, 0)),
            pl.BlockSpec((blk, 1), lambda i: (i, 0)),
        ],
        out_shape=[
            jax.ShapeDtypeStruct((n, out_dim), jnp.float32),
            jax.ShapeDtypeStruct((n, 1), jnp.float32),
            jax.ShapeDtypeStruct((n, 1), jnp.float32),
        ],
    )(x, W_l, A1, A2)

    mesh = plsc.VectorSubcoreMesh(core_axis_name="c", subcore_axis_name="s",
                                  num_cores=NC, num_subcores=NS)
    cp = pltpu.CompilerParams()
    if "needs_layout_passes" in pltpu.CompilerParams.__dataclass_fields__:
        cp = dataclasses.replace(cp, needs_layout_passes=False)
    edge_kernel = pl.kernel(
        functools.partial(_edge_body, n=n, n_pad=n_pad, d=out_dim, epw=epw),
        out_type=[
            jax.ShapeDtypeStruct((NC, n_pad, out_dim), jnp.float32),
            jax.ShapeDtypeStruct((NC, n_pad), jnp.float32),
        ],
        mesh=mesh,
        scratch_types=[
            pltpu.VMEM_SHARED((n_pad, out_dim), jnp.float32),
            pltpu.VMEM_SHARED((n_pad,), jnp.float32),
            pltpu.VMEM((8, CH), jnp.int32),
            pltpu.VMEM((8, CH), jnp.float32),
            pltpu.VMEM((3, CH, out_dim), jnp.float32),
            pltpu.VMEM((2, CH), jnp.float32),
            pltpu.SemaphoreType.DMA((4,)),
            pltpu.SemaphoreType.DMA((4,)),
            pltpu.SemaphoreType.DMA((3,)),
            pltpu.SemaphoreType.DMA((3,)),
        ],
        compiler_params=cp,
    )
    acc, den = edge_kernel(xl, s1.reshape(n), s2.reshape(n), row, col)

    out = pl.pallas_call(
        _out_body,
        grid=(grid,),
        in_specs=[
            pl.BlockSpec((NC, blk, out_dim), lambda i: (0, i, 0)),
            pl.BlockSpec((NC, blk, 1), lambda i: (0, i, 0)),
            pl.BlockSpec((blk, d), lambda i: (i, 0)),
            pl.BlockSpec((out_dim, d), lambda i: (0, 0)),
            pl.BlockSpec((1, out_dim), lambda i: (0, 0)),
        ],
        out_specs=pl.BlockSpec((blk, out_dim), lambda i: (i, 0)),
        out_shape=jax.ShapeDtypeStruct((n, out_dim), jnp.float32),
    )(acc, den.reshape(NC, n_pad, 1), x, W_r, bias.reshape(1, out_dim))
    return out
